# baseline (speedup 1.0000x reference)
import functools

import jax
import jax.numpy as jnp
from jax import lax
from jax.experimental import pallas as pl
from jax.experimental.pallas import tpu as pltpu
from jax.experimental.pallas import tpu_sc as plsc

E = 8
K = 2
D = 1024
DF = 1408
T = 2048
A = T * K

BLK = 512
NB = A // BLK + E
A_MAX = NB * BLK

NC = 2
NS = 16
NW = NC * NS
SCH = 64
TPW = T // NW
CCH = 16


@functools.cache
def _mesh():
    return plsc.VectorSubcoreMesh(core_axis_name="c", subcore_axis_name="s")


def _cumlanes(v):
    rows, n = v.shape
    lane = lax.broadcasted_iota(jnp.int32, (rows, n), 1)
    s = 1
    while s < n:
        v = v + jnp.where(lane >= s, pltpu.roll(v, s, 1), 0)
        s *= 2
    return v


def _route_body(x_ref, gw_ref, b_ref, pos_ref, w_ref, meta_ref):
    x = x_ref[...]
    gw = gw_ref[...]
    gates = lax.dot_general(gw, x, (((1,), (1,)), ((), ())),
                            preferred_element_type=jnp.float32)
    scores = jax.nn.sigmoid(gates)
    adj = scores + b_ref[...].reshape(E, 1)
    eidx = lax.broadcasted_iota(jnp.int32, (E, T), 0)
    m1 = jnp.max(adj, axis=0, keepdims=True)
    a1 = jnp.min(jnp.where(adj == m1, eidx, E), axis=0, keepdims=True)
    oh1 = eidx == a1
    adj2 = jnp.where(oh1, -jnp.inf, adj)
    m2 = jnp.max(adj2, axis=0, keepdims=True)
    a2 = jnp.min(jnp.where(adj2 == m2, eidx, E), axis=0, keepdims=True)
    oh2 = eidx == a2
    s1 = jnp.sum(jnp.where(oh1, scores, 0.0), axis=0, keepdims=True)
    s2 = jnp.sum(jnp.where(oh2, scores, 0.0), axis=0, keepdims=True)
    denom = s1 + s2 + 1e-20
    w_ref[...] = jnp.concatenate([s1 / denom, s2 / denom], axis=0)

    i1 = oh1.astype(jnp.int32)
    i2 = oh2.astype(jnp.int32)
    c1 = _cumlanes(i1)
    c2 = _cumlanes(i2)
    cnt1 = c1[:, T - 1:T]
    cnt2 = c2[:, T - 1:T]
    counts = cnt1 + cnt2
    padded = ((counts + BLK - 1) // BLK) * BLK
    tril = (lax.broadcasted_iota(jnp.int32, (E, E), 0)
            >= lax.broadcasted_iota(jnp.int32, (E, E), 1)).astype(jnp.float32)
    ends = lax.dot_general(tril, padded.astype(jnp.float32),
                           (((1,), (0,)), ((), ())),
                           preferred_element_type=jnp.float32).astype(jnp.int32)
    start = ends - padded
    pos0 = (jnp.sum(i1 * (start + c1), axis=0, keepdims=True) - 1)
    pos1 = (jnp.sum(i2 * (start + cnt1 + c2), axis=0, keepdims=True) - 1)
    pos_ref[...] = jnp.concatenate([pos0, pos1], axis=0)

    bidx = lax.broadcasted_iota(jnp.int32, (1, 128), 1)
    bstart = bidx * BLK
    be = jnp.sum((ends <= bstart).astype(jnp.int32), axis=0, keepdims=True)
    be = jnp.minimum(be, E - 1)
    nb_used = ends[E - 1:E, :]
    active = (bstart < nb_used).astype(jnp.int32)
    rb = jnp.minimum(bidx, nb_used // BLK - 1)
    gvalid = (padded > 0).astype(jnp.int32)
    gi = lax.dot_general(tril, gvalid.astype(jnp.float32),
                         (((1,), (0,)), ((), ())),
                         preferred_element_type=jnp.float32
                         ).astype(jnp.int32) - gvalid
    eiota_r = lax.broadcasted_iota(jnp.int32, (E, E), 1)
    eiota_c = lax.broadcasted_iota(jnp.int32, (E, E), 0)
    cand = jnp.where((eiota_r > eiota_c)
                     & (jnp.transpose(gvalid).astype(bool)),
                     eiota_r, E + 1)
    nxt = jnp.min(cand, axis=1, keepdims=True)
    pf_e = jnp.where(nxt <= E - 1, nxt, -1)
    fb = start // BLK
    oh_b = (lax.broadcasted_iota(jnp.int32, (E, 128), 0)
            == jnp.broadcast_to(be, (E, 128))).astype(jnp.int32)
    bset = jnp.sum(oh_b * (gi % 2), axis=0, keepdims=True)
    isfirst = jnp.sum(oh_b * (jnp.broadcast_to(fb, (E, 128))
                              == jnp.broadcast_to(bidx, (E, 128))
                              ).astype(jnp.int32), axis=0, keepdims=True)
    pf_b = jnp.where(isfirst > 0,
                     jnp.sum(oh_b * pf_e, axis=0, keepdims=True), -1)
    meta_ref[...] = jnp.concatenate(
        [be, rb, active, bset, pf_b, jnp.zeros((3, 128), jnp.int32)], axis=0)


def _route(x, gate_w, bias):
    return pl.pallas_call(
        _route_body,
        out_shape=(
            jax.ShapeDtypeStruct((K, T), jnp.int32),
            jax.ShapeDtypeStruct((K, T), jnp.float32),
            jax.ShapeDtypeStruct((8, 128), jnp.int32),
        ),
    )(x, gate_w, bias)


DCH = 32


def _dispatch_body(x_hbm, pos_hbm, xs_hbm, idx_v, r0_v, r1_v,
                   si0, si1, so0, so1):
    wid = lax.axis_index("s") * NC + lax.axis_index("c")
    tb = (wid % NS) * (T // NS)
    pltpu.sync_copy(pos_hbm.at[wid], idx_v)
    rbuf = (r0_v, r1_v)
    sin = (si0, si1)
    sout = (so0, so1)
    lds = [None, None]
    sts = [None, None]
    for c in range(2):
        lds[c] = pltpu.async_copy(
            x_hbm.at[pl.ds(tb + c * DCH, DCH)], rbuf[c], sin[c])
    for c in range(4):
        p = c % 2
        lds[p].wait()
        sts[p] = pltpu.async_copy(rbuf[p], xs_hbm.at[idx_v.at[c]], sout[p])
        if c + 2 < 4:
            sts[p].wait()
            lds[p] = pltpu.async_copy(
                x_hbm.at[pl.ds(tb + (c + 2) * DCH, DCH)], rbuf[p], sin[p])
    sts[0].wait()
    sts[1].wait()


@functools.cache
def _dispatch_sc():
    return pl.kernel(
        _dispatch_body,
        out_type=jax.ShapeDtypeStruct((A_MAX, D), jnp.float32),
        mesh=_mesh(),
        scratch_types=[
            pltpu.VMEM((4, DCH), jnp.int32),
            pltpu.VMEM((DCH, D), jnp.float32),
            pltpu.VMEM((DCH, D), jnp.float32),
            pltpu.SemaphoreType.DMA,
            pltpu.SemaphoreType.DMA,
            pltpu.SemaphoreType.DMA,
            pltpu.SemaphoreType.DMA,
        ],
    )


def _ffn_body(be_ref, rb_ref, act_ref, set_ref, pf_ref,
              xs_ref, wg_hbm, wu_hbm, wd_hbm, ys_ref,
              wgA, wuA, wdA, wgB, wuB, wdB, semA, semB):
    b = pl.program_id(0)
    prev = be_ref[jnp.maximum(b - 1, 0)]
    changed = jnp.logical_or(b == 0, be_ref[b] != prev)
    act = act_ref[b] == 1
    cur = set_ref[b]
    pf = pf_ref[b]

    @pl.when(b == 0)
    def _boot():
        e0 = be_ref[0]
        pltpu.make_async_copy(wg_hbm.at[e0], wgA, semA).start()
        pltpu.make_async_copy(wu_hbm.at[e0], wuA, semA).start()
        pltpu.make_async_copy(wd_hbm.at[e0], wdA, semA).start()

    @pl.when((pf >= 0) & (cur == 0))
    def _pfB():
        pltpu.make_async_copy(wg_hbm.at[pf], wgB, semB).start()
        pltpu.make_async_copy(wu_hbm.at[pf], wuB, semB).start()
        pltpu.make_async_copy(wd_hbm.at[pf], wdB, semB).start()

    @pl.when((pf >= 0) & (cur == 1))
    def _pfA():
        pltpu.make_async_copy(wg_hbm.at[pf], wgA, semA).start()
        pltpu.make_async_copy(wu_hbm.at[pf], wuA, semA).start()
        pltpu.make_async_copy(wd_hbm.at[pf], wdA, semA).start()

    e = be_ref[b]

    @pl.when(changed & act & (cur == 0))
    def _drainA():
        pltpu.make_async_copy(wg_hbm.at[e], wgA, semA).wait()
        pltpu.make_async_copy(wu_hbm.at[e], wuA, semA).wait()
        pltpu.make_async_copy(wd_hbm.at[e], wdA, semA).wait()

    @pl.when(changed & act & (cur == 1))
    def _drainB():
        pltpu.make_async_copy(wg_hbm.at[e], wgB, semB).wait()
        pltpu.make_async_copy(wu_hbm.at[e], wuB, semB).wait()
        pltpu.make_async_copy(wd_hbm.at[e], wdB, semB).wait()

    def _ffn(wg_v, wu_v, wd_v):
        xb = xs_ref[...]
        g = lax.dot_general(xb, wg_v[...], (((1,), (1,)), ((), ())),
                            preferred_element_type=jnp.float32)
        u = lax.dot_general(xb, wu_v[...], (((1,), (1,)), ((), ())),
                            preferred_element_type=jnp.float32)
        h = (g * jax.nn.sigmoid(g)) * u
        ys_ref[...] = lax.dot_general(h, wd_v[...], (((1,), (1,)), ((), ())),
                                      preferred_element_type=jnp.float32)

    @pl.when(act & (cur == 0))
    def _computeA():
        _ffn(wgA, wuA, wdA)

    @pl.when(act & (cur == 1))
    def _computeB():
        _ffn(wgB, wuB, wdB)


def _grouped_ffn(xs, Wg, Wu, Wd, be, rb, act, bset, pf):
    grid_spec = pltpu.PrefetchScalarGridSpec(
        num_scalar_prefetch=5,
        grid=(NB,),
        in_specs=[
            pl.BlockSpec((BLK, D), lambda b, *refs: (refs[1][b], 0)),
            pl.BlockSpec(memory_space=pltpu.MemorySpace.HBM),
            pl.BlockSpec(memory_space=pltpu.MemorySpace.HBM),
            pl.BlockSpec(memory_space=pltpu.MemorySpace.HBM),
        ],
        out_specs=pl.BlockSpec((BLK, D), lambda b, *refs: (b, 0)),
        scratch_shapes=[
            pltpu.VMEM((DF, D), jnp.float32),
            pltpu.VMEM((DF, D), jnp.float32),
            pltpu.VMEM((D, DF), jnp.float32),
            pltpu.VMEM((DF, D), jnp.float32),
            pltpu.VMEM((DF, D), jnp.float32),
            pltpu.VMEM((D, DF), jnp.float32),
            pltpu.SemaphoreType.DMA,
            pltpu.SemaphoreType.DMA,
        ],
    )
    return pl.pallas_call(
        _ffn_body,
        grid_spec=grid_spec,
        out_shape=jax.ShapeDtypeStruct((A_MAX, D), jnp.float32),
    )(be, rb, act, bset, pf, xs, Wg, Wu, Wd)


NCCH = 4


def _combine_body(ys_hbm, pos_hbm, wb_hbm, out_hbm, idx_v, wb_v,
                  y0a, y1a, y0b, y1b, o0, o1, sg0, sg1, so0, so1):
    wid = lax.axis_index("s") * NC + lax.axis_index("c")
    tb = wid * TPW
    pltpu.sync_copy(pos_hbm.at[wid], idx_v)
    pltpu.sync_copy(wb_hbm.at[wid], wb_v)
    ybufs = ((y0a, y1a), (y0b, y1b))
    obufs = (o0, o1)
    gsems = (sg0, sg1)
    osems = (so0, so1)

    def gather(c, p):
        return (pltpu.async_copy(ys_hbm.at[idx_v.at[0, c]], ybufs[p][0],
                                 gsems[p]),
                pltpu.async_copy(ys_hbm.at[idx_v.at[1, c]], ybufs[p][1],
                                 gsems[p]))

    g = [gather(0, 0), gather(1, 1)]
    st = [None, None]
    for c in range(NCCH):
        p = c & 1
        y0_v, y1_v = ybufs[p]
        o_v = obufs[p]
        for cp in g[p]:
            cp.wait()
        if st[p] is not None:
            st[p].wait()

        @plsc.parallel_loop(0, CCH, step=1)
        def _token(j, c=c, w0s=wb_v, y0_v=y0_v, y1_v=y1_v, o_v=o_v):
            w0 = w0s[0, c * CCH + j]
            w1 = w0s[1, c * CCH + j]
            for q in range(D // 16):
                o_v[j, pl.ds(q * 16, 16)] = (
                    w0 * y0_v[j, pl.ds(q * 16, 16)]
                    + w1 * y1_v[j, pl.ds(q * 16, 16)])

        st[p] = pltpu.async_copy(o_v, out_hbm.at[pl.ds(tb + c * CCH, CCH)],
                                 osems[p])
        if c + 2 < NCCH:
            g[p] = gather(c + 2, p)
    st[0].wait()
    st[1].wait()


@functools.cache
def _combine_sc():
    return pl.kernel(
        _combine_body,
        out_type=jax.ShapeDtypeStruct((T, D), jnp.float32),
        mesh=_mesh(),
        scratch_types=[
            pltpu.VMEM((K, NCCH, CCH), jnp.int32),
            pltpu.VMEM((K, TPW, 16), jnp.float32),
            pltpu.VMEM((CCH, D), jnp.float32),
            pltpu.VMEM((CCH, D), jnp.float32),
            pltpu.VMEM((CCH, D), jnp.float32),
            pltpu.VMEM((CCH, D), jnp.float32),
            pltpu.VMEM((CCH, D), jnp.float32),
            pltpu.VMEM((CCH, D), jnp.float32),
            pltpu.SemaphoreType.DMA,
            pltpu.SemaphoreType.DMA,
            pltpu.SemaphoreType.DMA,
            pltpu.SemaphoreType.DMA,
        ],
    )


def kernel(x, gate_w, Wg, Wu, Wd, bias):
    pos2, w, meta = _route(x, gate_w, bias)
    pos = pos2.reshape(-1)
    be = meta[0, :NB]
    rb = meta[1, :NB]
    act = meta[2, :NB]
    bset = meta[3, :NB]
    pf = meta[4, :NB]
    pos_d = pos.reshape(NW, 4, DCH)
    xs = _dispatch_sc()(x, pos_d)
    ys = _grouped_ffn(xs, Wg, Wu, Wd, be, rb, act, bset, pf)
    pos_c = (pos.reshape(K, NW, NCCH, CCH)
             .transpose(1, 0, 2, 3))
    wb = jnp.broadcast_to(
        w.reshape(K, NW, TPW).transpose(1, 0, 2)[..., None],
        (NW, K, TPW, 16))
    return _combine_sc()(ys, pos_c, wb)


# speedup vs baseline: 1.0482x; 1.0482x over previous
import functools

import jax
import jax.numpy as jnp
from jax import lax
from jax.experimental import pallas as pl
from jax.experimental.pallas import tpu as pltpu
from jax.experimental.pallas import tpu_sc as plsc

E = 8
K = 2
D = 1024
DF = 1408
T = 2048
A = T * K

BLK = 256
NB = A // BLK + E
A_MAX = NB * BLK

NC = 2
NS = 16
NW = NC * NS
SCH = 64
TPW = T // NW
CCH = 16


@functools.cache
def _mesh():
    return plsc.VectorSubcoreMesh(core_axis_name="c", subcore_axis_name="s")


def _cumlanes(v):
    rows, n = v.shape
    lane = lax.broadcasted_iota(jnp.int32, (rows, n), 1)
    s = 1
    while s < n:
        v = v + jnp.where(lane >= s, pltpu.roll(v, s, 1), 0)
        s *= 2
    return v


def _route_body(x_ref, gw_ref, b_ref, pos_ref, w_ref, meta_ref):
    x = x_ref[...]
    gw = gw_ref[...]
    gates = lax.dot_general(gw, x, (((1,), (1,)), ((), ())),
                            preferred_element_type=jnp.float32)
    scores = jax.nn.sigmoid(gates)
    adj = scores + b_ref[...].reshape(E, 1)
    eidx = lax.broadcasted_iota(jnp.int32, (E, T), 0)
    m1 = jnp.max(adj, axis=0, keepdims=True)
    a1 = jnp.min(jnp.where(adj == m1, eidx, E), axis=0, keepdims=True)
    oh1 = eidx == a1
    adj2 = jnp.where(oh1, -jnp.inf, adj)
    m2 = jnp.max(adj2, axis=0, keepdims=True)
    a2 = jnp.min(jnp.where(adj2 == m2, eidx, E), axis=0, keepdims=True)
    oh2 = eidx == a2
    s1 = jnp.sum(jnp.where(oh1, scores, 0.0), axis=0, keepdims=True)
    s2 = jnp.sum(jnp.where(oh2, scores, 0.0), axis=0, keepdims=True)
    denom = s1 + s2 + 1e-20
    w_ref[...] = jnp.concatenate([s1 / denom, s2 / denom], axis=0)

    i1 = oh1.astype(jnp.int32)
    i2 = oh2.astype(jnp.int32)
    c1 = _cumlanes(i1)
    c2 = _cumlanes(i2)
    cnt1 = c1[:, T - 1:T]
    cnt2 = c2[:, T - 1:T]
    counts = cnt1 + cnt2
    padded = ((counts + BLK - 1) // BLK) * BLK
    tril = (lax.broadcasted_iota(jnp.int32, (E, E), 0)
            >= lax.broadcasted_iota(jnp.int32, (E, E), 1)).astype(jnp.float32)
    ends = lax.dot_general(tril, padded.astype(jnp.float32),
                           (((1,), (0,)), ((), ())),
                           preferred_element_type=jnp.float32).astype(jnp.int32)
    start = ends - padded
    pos0 = (jnp.sum(i1 * (start + c1), axis=0, keepdims=True) - 1)
    pos1 = (jnp.sum(i2 * (start + cnt1 + c2), axis=0, keepdims=True) - 1)
    pos_ref[...] = jnp.concatenate([pos0, pos1], axis=0)

    bidx = lax.broadcasted_iota(jnp.int32, (1, 128), 1)
    bstart = bidx * BLK
    be = jnp.sum((ends <= bstart).astype(jnp.int32), axis=0, keepdims=True)
    be = jnp.minimum(be, E - 1)
    nb_used = ends[E - 1:E, :]
    active = (bstart < nb_used).astype(jnp.int32)
    rb = jnp.minimum(bidx, nb_used // BLK - 1)
    gvalid = (padded > 0).astype(jnp.int32)
    gi = lax.dot_general(tril, gvalid.astype(jnp.float32),
                         (((1,), (0,)), ((), ())),
                         preferred_element_type=jnp.float32
                         ).astype(jnp.int32) - gvalid
    eiota_r = lax.broadcasted_iota(jnp.int32, (E, E), 1)
    eiota_c = lax.broadcasted_iota(jnp.int32, (E, E), 0)
    cand = jnp.where((eiota_r > eiota_c)
                     & (jnp.transpose(gvalid).astype(bool)),
                     eiota_r, E + 1)
    nxt = jnp.min(cand, axis=1, keepdims=True)
    pf_e = jnp.where(nxt <= E - 1, nxt, -1)
    fb = start // BLK
    oh_b = (lax.broadcasted_iota(jnp.int32, (E, 128), 0)
            == jnp.broadcast_to(be, (E, 128))).astype(jnp.int32)
    bset = jnp.sum(oh_b * (gi % 2), axis=0, keepdims=True)
    isfirst = jnp.sum(oh_b * (jnp.broadcast_to(fb, (E, 128))
                              == jnp.broadcast_to(bidx, (E, 128))
                              ).astype(jnp.int32), axis=0, keepdims=True)
    pf_b = jnp.where(isfirst > 0,
                     jnp.sum(oh_b * pf_e, axis=0, keepdims=True), -1)
    meta_ref[...] = jnp.concatenate(
        [be, rb, active, bset, pf_b, jnp.zeros((3, 128), jnp.int32)], axis=0)


def _route(x, gate_w, bias):
    return pl.pallas_call(
        _route_body,
        out_shape=(
            jax.ShapeDtypeStruct((K, T), jnp.int32),
            jax.ShapeDtypeStruct((K, T), jnp.float32),
            jax.ShapeDtypeStruct((8, 128), jnp.int32),
        ),
    )(x, gate_w, bias)


DCH = 32


def _dispatch_body(x_hbm, pos_hbm, xs_hbm, idx_v, r0_v, r1_v,
                   si0, si1, so0, so1):
    wid = lax.axis_index("s") * NC + lax.axis_index("c")
    tb = (wid % NS) * (T // NS)
    pltpu.sync_copy(pos_hbm.at[wid], idx_v)
    rbuf = (r0_v, r1_v)
    sin = (si0, si1)
    sout = (so0, so1)
    lds = [None, None]
    sts = [None, None]
    for c in range(2):
        lds[c] = pltpu.async_copy(
            x_hbm.at[pl.ds(tb + c * DCH, DCH)], rbuf[c], sin[c])
    for c in range(4):
        p = c % 2
        lds[p].wait()
        sts[p] = pltpu.async_copy(rbuf[p], xs_hbm.at[idx_v.at[c]], sout[p])
        if c + 2 < 4:
            sts[p].wait()
            lds[p] = pltpu.async_copy(
                x_hbm.at[pl.ds(tb + (c + 2) * DCH, DCH)], rbuf[p], sin[p])
    sts[0].wait()
    sts[1].wait()


@functools.cache
def _dispatch_sc():
    return pl.kernel(
        _dispatch_body,
        out_type=jax.ShapeDtypeStruct((A_MAX, D), jnp.float32),
        mesh=_mesh(),
        scratch_types=[
            pltpu.VMEM((4, DCH), jnp.int32),
            pltpu.VMEM((DCH, D), jnp.float32),
            pltpu.VMEM((DCH, D), jnp.float32),
            pltpu.SemaphoreType.DMA,
            pltpu.SemaphoreType.DMA,
            pltpu.SemaphoreType.DMA,
            pltpu.SemaphoreType.DMA,
        ],
    )


def _ffn_body(be_ref, rb_ref, act_ref, set_ref, pf_ref,
              xs_ref, wg_hbm, wu_hbm, wd_hbm, ys_ref,
              wgA, wuA, wdA, wgB, wuB, wdB, semA, semB):
    b = pl.program_id(0)
    prev = be_ref[jnp.maximum(b - 1, 0)]
    changed = jnp.logical_or(b == 0, be_ref[b] != prev)
    act = act_ref[b] == 1
    cur = set_ref[b]
    pf = pf_ref[b]

    @pl.when(b == 0)
    def _boot():
        e0 = be_ref[0]
        pltpu.make_async_copy(wg_hbm.at[e0], wgA, semA).start()
        pltpu.make_async_copy(wu_hbm.at[e0], wuA, semA).start()
        pltpu.make_async_copy(wd_hbm.at[e0], wdA, semA).start()

    @pl.when((pf >= 0) & (cur == 0))
    def _pfB():
        pltpu.make_async_copy(wg_hbm.at[pf], wgB, semB).start()
        pltpu.make_async_copy(wu_hbm.at[pf], wuB, semB).start()
        pltpu.make_async_copy(wd_hbm.at[pf], wdB, semB).start()

    @pl.when((pf >= 0) & (cur == 1))
    def _pfA():
        pltpu.make_async_copy(wg_hbm.at[pf], wgA, semA).start()
        pltpu.make_async_copy(wu_hbm.at[pf], wuA, semA).start()
        pltpu.make_async_copy(wd_hbm.at[pf], wdA, semA).start()

    e = be_ref[b]

    @pl.when(changed & act & (cur == 0))
    def _drainA():
        pltpu.make_async_copy(wg_hbm.at[e], wgA, semA).wait()
        pltpu.make_async_copy(wu_hbm.at[e], wuA, semA).wait()
        pltpu.make_async_copy(wd_hbm.at[e], wdA, semA).wait()

    @pl.when(changed & act & (cur == 1))
    def _drainB():
        pltpu.make_async_copy(wg_hbm.at[e], wgB, semB).wait()
        pltpu.make_async_copy(wu_hbm.at[e], wuB, semB).wait()
        pltpu.make_async_copy(wd_hbm.at[e], wdB, semB).wait()

    def _ffn(wg_v, wu_v, wd_v):
        xb = xs_ref[...]
        g = lax.dot_general(xb, wg_v[...], (((1,), (1,)), ((), ())),
                            preferred_element_type=jnp.float32)
        u = lax.dot_general(xb, wu_v[...], (((1,), (1,)), ((), ())),
                            preferred_element_type=jnp.float32)
        h = (g * jax.nn.sigmoid(g)) * u
        ys_ref[...] = lax.dot_general(h, wd_v[...], (((1,), (1,)), ((), ())),
                                      preferred_element_type=jnp.float32)

    @pl.when(act & (cur == 0))
    def _computeA():
        _ffn(wgA, wuA, wdA)

    @pl.when(act & (cur == 1))
    def _computeB():
        _ffn(wgB, wuB, wdB)


def _grouped_ffn(xs, Wg, Wu, Wd, be, rb, act, bset, pf):
    grid_spec = pltpu.PrefetchScalarGridSpec(
        num_scalar_prefetch=5,
        grid=(NB,),
        in_specs=[
            pl.BlockSpec((BLK, D), lambda b, *refs: (refs[1][b], 0)),
            pl.BlockSpec(memory_space=pltpu.MemorySpace.HBM),
            pl.BlockSpec(memory_space=pltpu.MemorySpace.HBM),
            pl.BlockSpec(memory_space=pltpu.MemorySpace.HBM),
        ],
        out_specs=pl.BlockSpec((BLK, D), lambda b, *refs: (b, 0)),
        scratch_shapes=[
            pltpu.VMEM((DF, D), jnp.float32),
            pltpu.VMEM((DF, D), jnp.float32),
            pltpu.VMEM((D, DF), jnp.float32),
            pltpu.VMEM((DF, D), jnp.float32),
            pltpu.VMEM((DF, D), jnp.float32),
            pltpu.VMEM((D, DF), jnp.float32),
            pltpu.SemaphoreType.DMA,
            pltpu.SemaphoreType.DMA,
        ],
    )
    return pl.pallas_call(
        _ffn_body,
        grid_spec=grid_spec,
        out_shape=jax.ShapeDtypeStruct((A_MAX, D), jnp.float32),
    )(be, rb, act, bset, pf, xs, Wg, Wu, Wd)


NCCH = 4


def _combine_body(ys_hbm, pos_hbm, wb_hbm, out_hbm, idx_v, wb_v,
                  y0a, y1a, y0b, y1b, o0, o1, sg0, sg1, so0, so1):
    wid = lax.axis_index("s") * NC + lax.axis_index("c")
    tb = wid * TPW
    pltpu.sync_copy(pos_hbm.at[wid], idx_v)
    pltpu.sync_copy(wb_hbm.at[wid], wb_v)
    ybufs = ((y0a, y1a), (y0b, y1b))
    obufs = (o0, o1)
    gsems = (sg0, sg1)
    osems = (so0, so1)

    def gather(c, p):
        return (pltpu.async_copy(ys_hbm.at[idx_v.at[0, c]], ybufs[p][0],
                                 gsems[p]),
                pltpu.async_copy(ys_hbm.at[idx_v.at[1, c]], ybufs[p][1],
                                 gsems[p]))

    g = [gather(0, 0), gather(1, 1)]
    st = [None, None]
    for c in range(NCCH):
        p = c & 1
        y0_v, y1_v = ybufs[p]
        o_v = obufs[p]
        for cp in g[p]:
            cp.wait()
        if st[p] is not None:
            st[p].wait()

        @plsc.parallel_loop(0, CCH, step=1)
        def _token(j, c=c, w0s=wb_v, y0_v=y0_v, y1_v=y1_v, o_v=o_v):
            w0 = w0s[0, c * CCH + j]
            w1 = w0s[1, c * CCH + j]
            for q in range(D // 16):
                o_v[j, pl.ds(q * 16, 16)] = (
                    w0 * y0_v[j, pl.ds(q * 16, 16)]
                    + w1 * y1_v[j, pl.ds(q * 16, 16)])

        st[p] = pltpu.async_copy(o_v, out_hbm.at[pl.ds(tb + c * CCH, CCH)],
                                 osems[p])
        if c + 2 < NCCH:
            g[p] = gather(c + 2, p)
    st[0].wait()
    st[1].wait()


@functools.cache
def _combine_sc():
    return pl.kernel(
        _combine_body,
        out_type=jax.ShapeDtypeStruct((T, D), jnp.float32),
        mesh=_mesh(),
        scratch_types=[
            pltpu.VMEM((K, NCCH, CCH), jnp.int32),
            pltpu.VMEM((K, TPW, 16), jnp.float32),
            pltpu.VMEM((CCH, D), jnp.float32),
            pltpu.VMEM((CCH, D), jnp.float32),
            pltpu.VMEM((CCH, D), jnp.float32),
            pltpu.VMEM((CCH, D), jnp.float32),
            pltpu.VMEM((CCH, D), jnp.float32),
            pltpu.VMEM((CCH, D), jnp.float32),
            pltpu.SemaphoreType.DMA,
            pltpu.SemaphoreType.DMA,
            pltpu.SemaphoreType.DMA,
            pltpu.SemaphoreType.DMA,
        ],
    )


def kernel(x, gate_w, Wg, Wu, Wd, bias):
    pos2, w, meta = _route(x, gate_w, bias)
    pos = pos2.reshape(-1)
    be = meta[0, :NB]
    rb = meta[1, :NB]
    act = meta[2, :NB]
    bset = meta[3, :NB]
    pf = meta[4, :NB]
    pos_d = pos.reshape(NW, 4, DCH)
    xs = _dispatch_sc()(x, pos_d)
    ys = _grouped_ffn(xs, Wg, Wu, Wd, be, rb, act, bset, pf)
    pos_c = (pos.reshape(K, NW, NCCH, CCH)
             .transpose(1, 0, 2, 3))
    wb = jnp.broadcast_to(
        w.reshape(K, NW, TPW).transpose(1, 0, 2)[..., None],
        (NW, K, TPW, 16))
    return _combine_sc()(ys, pos_c, wb)
